# Initial kernel scaffold; baseline (speedup 1.0000x reference)
import jax
import jax.numpy as jnp
from jax.experimental import pallas as pl


def kernel(features, edge_index, W1, b1, W2, b2):
    raise NotImplementedError("write your pallas kernel here")


# speedup vs baseline: 15.9479x; 15.9479x over previous
import functools

import jax
import jax.numpy as jnp
from jax import lax
from jax.experimental import pallas as pl
from jax.experimental.pallas import tpu as pltpu
from jax.experimental.pallas import tpu_sc as plsc

N_NODES = 10000
N_EDGES = 320000
D = 128

NC = 2
NS = 16
NW = NC * NS
E_PER_W = N_EDGES // NW
CHUNK = 80
NCHUNKS = E_PER_W // CHUNK
ACC_ROWS = 10240
ROWS_PER_TILE = ACC_ROWS // NS


def _prep_body(f_ref, w1_ref, w2_ref, b1_ref, p_ref, bf_ref):
    wf = jnp.dot(w1_ref[...], w2_ref[...], preferred_element_type=jnp.float32)
    p_ref[...] = jnp.dot(f_ref[...], wf, preferred_element_type=jnp.float32)
    bf_ref[...] = jnp.sum(b1_ref[...] * w2_ref[...], axis=0, keepdims=True)


def _prep(features, W1, b1, W2):
    return pl.pallas_call(
        _prep_body,
        out_shape=(
            jax.ShapeDtypeStruct((N_NODES, D), jnp.float32),
            jax.ShapeDtypeStruct((1, D), jnp.float32),
        ),
    )(features, W1, W2, b1.reshape(-1, 1))


def _combine_body(parts_ref, b_ref, o_ref):
    o_ref[...] = (parts_ref[0, :N_NODES] + parts_ref[1, :N_NODES]
                  + b_ref[...])


def _combine(parts, bias_row):
    return pl.pallas_call(
        _combine_body,
        out_shape=jax.ShapeDtypeStruct((N_NODES, D), jnp.float32),
    )(parts, bias_row)


def _agg_body(x_hbm, src_hbm, dst_hbm, out_hbm,
              src_v, dst_v, rows_v, acc_sh, sem):
    c = lax.axis_index("c")
    s = lax.axis_index("s")
    wid = s * NC + c

    zero16 = jnp.zeros((16,), jnp.float32)

    def zbody(i, carry):
        for j in range(D // 16):
            rows_v[i, pl.ds(j * 16, 16)] = zero16
        return carry

    lax.fori_loop(0, CHUNK, zbody, 0)
    for k in range(ROWS_PER_TILE // CHUNK):
        pltpu.sync_copy(
            rows_v, acc_sh.at[pl.ds(s * ROWS_PER_TILE + k * CHUNK, CHUNK)])
    plsc.subcore_barrier()

    pltpu.sync_copy(src_hbm.at[wid], src_v)
    pltpu.sync_copy(dst_hbm.at[wid], dst_v)

    def body(ci, carry):
        pltpu.async_copy(x_hbm.at[src_v.at[ci]], rows_v, sem).wait()
        pltpu.sync_copy(rows_v, acc_sh.at[dst_v.at[ci]], add=True)
        return carry

    lax.fori_loop(0, NCHUNKS, body, 0)
    plsc.subcore_barrier()

    pltpu.sync_copy(acc_sh.at[pl.ds(s * ROWS_PER_TILE, ROWS_PER_TILE)],
                    out_hbm.at[c, pl.ds(s * ROWS_PER_TILE, ROWS_PER_TILE)])


@functools.partial(
    pl.kernel,
    out_type=jax.ShapeDtypeStruct((NC, ACC_ROWS, D), jnp.float32),
    mesh=plsc.VectorSubcoreMesh(core_axis_name="c", subcore_axis_name="s"),
    scratch_types=[
        pltpu.VMEM((NCHUNKS, CHUNK), jnp.int32),
        pltpu.VMEM((NCHUNKS, CHUNK), jnp.int32),
        pltpu.VMEM((CHUNK, D), jnp.float32),
        pltpu.VMEM_SHARED((ACC_ROWS, D), jnp.float32),
        pltpu.SemaphoreType.DMA,
    ],
)
def _agg(x_hbm, src_hbm, dst_hbm, out_hbm,
         src_v, dst_v, rows_v, acc_sh, sem):
    _agg_body(x_hbm, src_hbm, dst_hbm, out_hbm,
              src_v, dst_v, rows_v, acc_sh, sem)


def kernel(features, edge_index, W1, b1, W2, b2):
    ei = edge_index.astype(jnp.int32)
    src3 = ei[0].reshape(NW, NCHUNKS, CHUNK)
    dst3 = ei[1].reshape(NW, NCHUNKS, CHUNK)

    p, bf = _prep(features, W1, b1, W2)
    parts1 = _agg(p, src3, dst3)
    g = _combine(parts1, bf)
    parts2 = _agg(g, src3, dst3)
    out = _combine(parts2, b2.reshape(1, D))
    return out
